# baseline (speedup 1.0000x reference)
import jax
import jax.numpy as jnp
from jax import lax
from jax.experimental import pallas as pl
from jax.experimental.pallas import tpu as pltpu
from jax.experimental.pallas import tpu_sc as plsc

PROMPT_NUM = 50
HIDDEN_SIZE = 1024
BATCH = 1024

_NBUF = 8


def _tc_body(table_v, emb_hbm, mask_hbm, staged, ones_v, sems, mask_sem):
    ones_v[...] = jnp.ones((PROMPT_NUM, BATCH), jnp.float32)
    mask_h = pltpu.make_async_copy(ones_v, mask_hbm, mask_sem)
    mask_h.start()
    handles = []
    for p in range(PROMPT_NUM):
        s = p % _NBUF
        if p >= _NBUF:
            handles[p - _NBUF].wait()
        staged[s, ...] = jnp.broadcast_to(
            table_v[pl.ds(p, 1), :], (BATCH, HIDDEN_SIZE)
        )
        h = pltpu.make_async_copy(staged.at[s], emb_hbm.at[p], sems.at[s])
        h.start()
        handles.append(h)
    for p in range(PROMPT_NUM - _NBUF, PROMPT_NUM):
        handles[p].wait()
    mask_h.wait()


def _tc_broadcast(prompt_table):
    return pl.pallas_call(
        _tc_body,
        out_shape=(
            jax.ShapeDtypeStruct((PROMPT_NUM, BATCH, HIDDEN_SIZE), jnp.float32),
            jax.ShapeDtypeStruct((PROMPT_NUM, BATCH), jnp.float32),
        ),
        in_specs=[pl.BlockSpec(memory_space=pltpu.VMEM)],
        out_specs=(
            pl.BlockSpec(memory_space=pl.ANY),
            pl.BlockSpec(memory_space=pl.ANY),
        ),
        scratch_shapes=[
            pltpu.VMEM((_NBUF, BATCH, HIDDEN_SIZE), jnp.float32),
            pltpu.VMEM((PROMPT_NUM, BATCH), jnp.float32),
            pltpu.SemaphoreType.DMA((_NBUF,)),
            pltpu.SemaphoreType.DMA,
        ],
    )(prompt_table)


def kernel(batch_size, prompt_table):
    emb_t, mask_t = _tc_broadcast(prompt_table)
    emb = jnp.transpose(emb_t, (1, 0, 2))
    mask = jnp.transpose(mask_t, (1, 0))
    return emb, mask


# speedup vs baseline: 1.0095x; 1.0095x over previous
import jax
import jax.numpy as jnp
from jax import lax
from jax.experimental import pallas as pl
from jax.experimental.pallas import tpu as pltpu
from jax.experimental.pallas import tpu_sc as plsc

PROMPT_NUM = 50
HIDDEN_SIZE = 1024
BATCH = 1024

_PP = 2
_NSTEP = PROMPT_NUM // _PP
_NBUF = 4


def _tc_body(table_v, emb_hbm, mask_hbm, staged, ones_v, sems, mask_sem):
    handles = []
    mask_started = False
    for t in range(_NSTEP):
        s = t % _NBUF
        if t >= _NBUF:
            handles[t - _NBUF].wait()
        staged[s, ...] = jnp.broadcast_to(
            table_v[pl.ds(t * _PP, _PP), :][:, None, :],
            (_PP, BATCH, HIDDEN_SIZE),
        )
        h = pltpu.make_async_copy(
            staged.at[s], emb_hbm.at[pl.ds(t * _PP, _PP)], sems.at[s]
        )
        h.start()
        handles.append(h)
        if not mask_started:
            ones_v[...] = jnp.ones((PROMPT_NUM, BATCH), jnp.float32)
            mask_h = pltpu.make_async_copy(ones_v, mask_hbm, mask_sem)
            mask_h.start()
            mask_started = True
    for t in range(_NSTEP - _NBUF, _NSTEP):
        handles[t].wait()
    mask_h.wait()


def _tc_broadcast(prompt_table):
    return pl.pallas_call(
        _tc_body,
        out_shape=(
            jax.ShapeDtypeStruct((PROMPT_NUM, BATCH, HIDDEN_SIZE), jnp.float32),
            jax.ShapeDtypeStruct((PROMPT_NUM, BATCH), jnp.float32),
        ),
        in_specs=[pl.BlockSpec(memory_space=pltpu.VMEM)],
        out_specs=(
            pl.BlockSpec(memory_space=pl.ANY),
            pl.BlockSpec(memory_space=pl.ANY),
        ),
        scratch_shapes=[
            pltpu.VMEM((_NBUF, _PP, BATCH, HIDDEN_SIZE), jnp.float32),
            pltpu.VMEM((PROMPT_NUM, BATCH), jnp.float32),
            pltpu.SemaphoreType.DMA((_NBUF,)),
            pltpu.SemaphoreType.DMA,
        ],
    )(prompt_table)


def kernel(batch_size, prompt_table):
    emb_t, mask_t = _tc_broadcast(prompt_table)
    emb = jnp.transpose(emb_t, (1, 0, 2))
    mask = jnp.transpose(mask_t, (1, 0))
    return emb, mask
